# baseline (speedup 1.0000x reference)
import functools

import jax
import jax.numpy as jnp
from jax.experimental import pallas as pl
from jax.experimental.pallas import tpu as pltpu

EPS = 1.1920929e-07

BL = 512
BM = 512
BN = 512


def _linear_kernel(x_ref, w_ref, b_ref, gl_ref):
    gl_ref[...] = jnp.maximum(
        jnp.dot(x_ref[...], w_ref[...], preferred_element_type=jnp.float32)
        + b_ref[...],
        0.0,
    )


def _prob_kernel(glr_ref, glc_ref, prob_ref, rs_ref):
    i = pl.program_id(0)
    j = pl.program_id(1)
    glr = glr_ref[...]
    glc = glc_ref[...]
    sq_r = jnp.sum(glr * glr, axis=1, keepdims=True)
    sq_c = jnp.sum(glc * glc, axis=1, keepdims=True).T
    gram = jax.lax.dot_general(
        glr, glc, (((1,), (1,)), ((), ())),
        preferred_element_type=jnp.float32,
    )
    diff = jnp.maximum(sq_r + sq_c - 2.0 * gram, 0.0)
    rows = i * BM + jax.lax.broadcasted_iota(jnp.int32, (BM, BN), 0)
    cols = j * BN + jax.lax.broadcasted_iota(jnp.int32, (BM, BN), 1)
    diff = jnp.where(rows == cols, 0.0, diff)
    prob = jnp.where(diff == 0.0, 1.0, jnp.exp(-jnp.sqrt(diff + EPS)))
    prob_ref[...] = prob

    @pl.when(j == 0)
    def _():
        rs_ref[...] = jnp.full_like(rs_ref[...], -1.0)

    rs_ref[...] += jnp.sum(prob, axis=1, keepdims=True)


def _dinv(deg):
    return jnp.where(deg > 0.0, jax.lax.rsqrt(deg), 0.0)


def _tx1_kernel(prob_ref, xj_ref, xi_ref, degj_ref, degi_ref, tx1_ref, *, nj):
    j = pl.program_id(1)
    u_j = _dinv(degj_ref[...]) * xj_ref[...]

    @pl.when(j == 0)
    def _():
        tx1_ref[...] = -(_dinv(degi_ref[...]) * xi_ref[...])

    tx1_ref[...] += jnp.dot(prob_ref[...], u_j, preferred_element_type=jnp.float32)

    @pl.when(j == nj - 1)
    def _():
        tx1_ref[...] = -(_dinv(degi_ref[...]) * tx1_ref[...])


def _out_kernel(prob_ref, tx1j_ref, tx1i_ref, xi_ref, degj_ref, degi_ref,
                w_ref, b_ref, out_ref, acc_ref, *, nj):
    j = pl.program_id(1)
    u_j = _dinv(degj_ref[...]) * tx1j_ref[...]

    @pl.when(j == 0)
    def _():
        acc_ref[...] = -(_dinv(degi_ref[...]) * tx1i_ref[...])

    acc_ref[...] += jnp.dot(prob_ref[...], u_j, preferred_element_type=jnp.float32)

    @pl.when(j == nj - 1)
    def _():
        lt = -(_dinv(degi_ref[...]) * acc_ref[...])
        xi = xi_ref[...]
        tx2 = 2.0 * lt - xi
        out = jnp.dot(xi, w_ref[0], preferred_element_type=jnp.float32)
        out += jnp.dot(tx1i_ref[...], w_ref[1], preferred_element_type=jnp.float32)
        out += jnp.dot(tx2, w_ref[2], preferred_element_type=jnp.float32)
        out_ref[...] = jnp.maximum(out + b_ref[...], 0.0)


def kernel(input, adj, lin_w, lin_b, cheb_w, cheb_b):
    x = input
    n, d = x.shape
    ni, nj = n // BM, n // BN

    gl = pl.pallas_call(
        _linear_kernel,
        grid=(n // BL,),
        in_specs=[
            pl.BlockSpec((BL, d), lambda i: (i, 0)),
            pl.BlockSpec((d, d), lambda i: (0, 0)),
            pl.BlockSpec((1, d), lambda i: (0, 0)),
        ],
        out_specs=pl.BlockSpec((BL, d), lambda i: (i, 0)),
        out_shape=jax.ShapeDtypeStruct((n, d), jnp.float32),
    )(x, lin_w, lin_b.reshape(1, d))

    prob, deg = pl.pallas_call(
        _prob_kernel,
        grid=(ni, nj),
        in_specs=[
            pl.BlockSpec((BM, d), lambda i, j: (i, 0)),
            pl.BlockSpec((BN, d), lambda i, j: (j, 0)),
        ],
        out_specs=[
            pl.BlockSpec((BM, BN), lambda i, j: (i, j)),
            pl.BlockSpec((BM, 1), lambda i, j: (i, 0)),
        ],
        out_shape=[
            jax.ShapeDtypeStruct((n, n), jnp.float32),
            jax.ShapeDtypeStruct((n, 1), jnp.float32),
        ],
    )(gl, gl)

    tx1 = pl.pallas_call(
        functools.partial(_tx1_kernel, nj=nj),
        grid=(ni, nj),
        in_specs=[
            pl.BlockSpec((BM, BN), lambda i, j: (i, j)),
            pl.BlockSpec((BN, d), lambda i, j: (j, 0)),
            pl.BlockSpec((BM, d), lambda i, j: (i, 0)),
            pl.BlockSpec((BN, 1), lambda i, j: (j, 0)),
            pl.BlockSpec((BM, 1), lambda i, j: (i, 0)),
        ],
        out_specs=pl.BlockSpec((BM, d), lambda i, j: (i, 0)),
        out_shape=jax.ShapeDtypeStruct((n, d), jnp.float32),
    )(prob, x, x, deg, deg)

    out = pl.pallas_call(
        functools.partial(_out_kernel, nj=nj),
        grid=(ni, nj),
        in_specs=[
            pl.BlockSpec((BM, BN), lambda i, j: (i, j)),
            pl.BlockSpec((BN, d), lambda i, j: (j, 0)),
            pl.BlockSpec((BM, d), lambda i, j: (i, 0)),
            pl.BlockSpec((BM, d), lambda i, j: (i, 0)),
            pl.BlockSpec((BN, 1), lambda i, j: (j, 0)),
            pl.BlockSpec((BM, 1), lambda i, j: (i, 0)),
            pl.BlockSpec((3, d, d), lambda i, j: (0, 0, 0)),
            pl.BlockSpec((1, d), lambda i, j: (0, 0)),
        ],
        out_specs=pl.BlockSpec((BM, d), lambda i, j: (i, 0)),
        out_shape=jax.ShapeDtypeStruct((n, d), jnp.float32),
        scratch_shapes=[pltpu.VMEM((BM, d), jnp.float32)],
    )(prob, tx1, tx1, x, deg, deg, cheb_w, cheb_b.reshape(1, d))

    return out, prob


# speedup vs baseline: 1.4492x; 1.4492x over previous
import functools

import jax
import jax.numpy as jnp
from jax.experimental import pallas as pl
from jax.experimental.pallas import tpu as pltpu

EPS = 1.1920929e-07

BL = 512
BM = 512
BN = 512
BK = 1024


def _linear_kernel(x_ref, w_ref, b_ref, glh_ref, sq_ref):
    gl = jnp.maximum(
        jnp.dot(x_ref[...], w_ref[...], preferred_element_type=jnp.float32)
        + b_ref[...],
        0.0,
    )
    glh_ref[...] = gl.astype(jnp.bfloat16)
    sq_ref[...] = jnp.sum(gl * gl, axis=1, keepdims=True)


def _prob_kernel(glh_ref, sq_ref, sqt_ref, prob_ref, pb_ref, rs_ref):
    i = pl.program_id(0)
    j = pl.program_id(1)
    glr = glh_ref[pl.ds(i * BM, BM), :]
    glc = glh_ref[pl.ds(j * BN, BN), :]
    sq_r = sq_ref[...]
    sq_c = sqt_ref[...]
    g2 = jax.lax.dot_general(
        glr * jnp.bfloat16(-2.0), glc, (((1,), (1,)), ((), ())),
        preferred_element_type=jnp.float32,
    )
    diff = jnp.maximum((g2 + sq_c) + sq_r, 0.0)
    prob = jnp.exp(-jnp.sqrt(diff + EPS))

    @pl.when(i == j)
    def _():
        rloc = jax.lax.broadcasted_iota(jnp.int32, (BM, BN), 0)
        cloc = jax.lax.broadcasted_iota(jnp.int32, (BM, BN), 1)
        pfix = jnp.where(rloc == cloc, 1.0, prob)
        prob_ref[...] = pfix
        pb_ref[...] = pfix.astype(jnp.bfloat16)

    @pl.when(i != j)
    def _():
        prob_ref[...] = prob
        pb_ref[...] = prob.astype(jnp.bfloat16)

    pv = prob_ref[...]
    ps = (pv[:, 0:128] + pv[:, 128:256]) + (pv[:, 256:384] + pv[:, 384:512])

    @pl.when(j == 0)
    def _():
        rs_ref[...] = ps

    @pl.when(j != 0)
    def _():
        rs_ref[...] += ps


def _dinv_kernel(rs_ref, x_ref, dinv_ref, uh_ref):
    deg = jnp.sum(rs_ref[...], axis=1, keepdims=True) - 1.0
    dinv = jnp.where(deg > 0.0, jax.lax.rsqrt(deg), 0.0)
    dinv_ref[...] = dinv
    uh_ref[...] = (dinv * x_ref[...]).astype(jnp.bfloat16)


def _tx1_kernel(pb_ref, uh_ref, dinv_ref, tx1_ref, u2h_ref, *, nj):
    j = pl.program_id(1)
    u_j = uh_ref[pl.ds(j * BK, BK), :]

    @pl.when(j == 0)
    def _():
        tx1_ref[...] = -uh_ref[pl.ds(pl.program_id(0) * BM, BM), :].astype(jnp.float32)

    tx1_ref[...] += jax.lax.dot_general(
        pb_ref[...], u_j, (((1,), (0,)), ((), ())),
        preferred_element_type=jnp.float32,
    )

    @pl.when(j == nj - 1)
    def _():
        dinv_i = dinv_ref[...]
        tx1 = -(dinv_i * tx1_ref[...])
        tx1_ref[...] = tx1
        u2h_ref[...] = (dinv_i * tx1).astype(jnp.bfloat16)


def _out_kernel(pb_ref, u2h_ref, tx1_ref, x_ref, dinv_ref, w_ref, b_ref,
                out_ref, acc_ref, *, nj):
    i = pl.program_id(0)
    j = pl.program_id(1)
    u2_j = u2h_ref[pl.ds(j * BK, BK), :]

    @pl.when(j == 0)
    def _():
        acc_ref[...] = -u2h_ref[pl.ds(i * BM, BM), :].astype(jnp.float32)

    acc_ref[...] += jax.lax.dot_general(
        pb_ref[...], u2_j, (((1,), (0,)), ((), ())),
        preferred_element_type=jnp.float32,
    )

    @pl.when(j == nj - 1)
    def _():
        lt = -(dinv_ref[...] * acc_ref[...])
        xi = x_ref[pl.ds(i * BM, BM), :]
        tx1i = tx1_ref[pl.ds(i * BM, BM), :]
        tx2 = 2.0 * lt - xi
        out = jnp.dot(xi, w_ref[0], preferred_element_type=jnp.float32)
        out += jnp.dot(tx1i, w_ref[1], preferred_element_type=jnp.float32)
        out += jnp.dot(tx2, w_ref[2], preferred_element_type=jnp.float32)
        out_ref[...] = jnp.maximum(out + b_ref[...], 0.0)


def kernel(input, adj, lin_w, lin_b, cheb_w, cheb_b):
    x = input
    n, d = x.shape
    ni, nj = n // BM, n // BN
    njk = n // BK

    glh, sq = pl.pallas_call(
        _linear_kernel,
        grid=(n // BL,),
        in_specs=[
            pl.BlockSpec((BL, d), lambda i: (i, 0)),
            pl.BlockSpec((d, d), lambda i: (0, 0)),
            pl.BlockSpec((1, d), lambda i: (0, 0)),
        ],
        out_specs=[
            pl.BlockSpec((BL, d), lambda i: (i, 0)),
            pl.BlockSpec((BL, 1), lambda i: (i, 0)),
        ],
        out_shape=[
            jax.ShapeDtypeStruct((n, d), jnp.bfloat16),
            jax.ShapeDtypeStruct((n, 1), jnp.float32),
        ],
    )(x, lin_w, lin_b.reshape(1, d))

    sqt = sq.reshape(1, n)

    prob, pb, rs = pl.pallas_call(
        _prob_kernel,
        grid=(ni, nj),
        in_specs=[
            pl.BlockSpec((n, d), lambda i, j: (0, 0)),
            pl.BlockSpec((BM, 1), lambda i, j: (i, 0)),
            pl.BlockSpec((1, BN), lambda i, j: (0, j)),
        ],
        out_specs=[
            pl.BlockSpec((BM, BN), lambda i, j: (i, j)),
            pl.BlockSpec((BM, BN), lambda i, j: (i, j)),
            pl.BlockSpec((BM, 128), lambda i, j: (i, 0)),
        ],
        out_shape=[
            jax.ShapeDtypeStruct((n, n), jnp.float32),
            jax.ShapeDtypeStruct((n, n), jnp.bfloat16),
            jax.ShapeDtypeStruct((n, 128), jnp.float32),
        ],
    )(glh, sq, sqt)

    dinv, uh = pl.pallas_call(
        _dinv_kernel,
        grid=(ni,),
        in_specs=[
            pl.BlockSpec((BM, 128), lambda i: (i, 0)),
            pl.BlockSpec((BM, d), lambda i: (i, 0)),
        ],
        out_specs=[
            pl.BlockSpec((BM, 1), lambda i: (i, 0)),
            pl.BlockSpec((BM, d), lambda i: (i, 0)),
        ],
        out_shape=[
            jax.ShapeDtypeStruct((n, 1), jnp.float32),
            jax.ShapeDtypeStruct((n, d), jnp.bfloat16),
        ],
    )(rs, x)

    tx1, u2h = pl.pallas_call(
        functools.partial(_tx1_kernel, nj=njk),
        grid=(ni, njk),
        in_specs=[
            pl.BlockSpec((BM, BK), lambda i, j: (i, j)),
            pl.BlockSpec((n, d), lambda i, j: (0, 0)),
            pl.BlockSpec((BM, 1), lambda i, j: (i, 0)),
        ],
        out_specs=[
            pl.BlockSpec((BM, d), lambda i, j: (i, 0)),
            pl.BlockSpec((BM, d), lambda i, j: (i, 0)),
        ],
        out_shape=[
            jax.ShapeDtypeStruct((n, d), jnp.float32),
            jax.ShapeDtypeStruct((n, d), jnp.bfloat16),
        ],
    )(pb, uh, dinv)

    out = pl.pallas_call(
        functools.partial(_out_kernel, nj=njk),
        grid=(ni, njk),
        in_specs=[
            pl.BlockSpec((BM, BK), lambda i, j: (i, j)),
            pl.BlockSpec((n, d), lambda i, j: (0, 0)),
            pl.BlockSpec((n, d), lambda i, j: (0, 0)),
            pl.BlockSpec((n, d), lambda i, j: (0, 0)),
            pl.BlockSpec((BM, 1), lambda i, j: (i, 0)),
            pl.BlockSpec((3, d, d), lambda i, j: (0, 0, 0)),
            pl.BlockSpec((1, d), lambda i, j: (0, 0)),
        ],
        out_specs=pl.BlockSpec((BM, d), lambda i, j: (i, 0)),
        out_shape=jax.ShapeDtypeStruct((n, d), jnp.float32),
        scratch_shapes=[pltpu.VMEM((BM, d), jnp.float32)],
    )(pb, u2h, tx1, x, dinv, cheb_w, cheb_b.reshape(1, d))

    return out, prob


# speedup vs baseline: 1.5977x; 1.1025x over previous
import functools

import jax
import jax.numpy as jnp
from jax.experimental import pallas as pl
from jax.experimental.pallas import tpu as pltpu

EPS = 1.1920929e-07

BL = 512
BM = 512
BN = 1024
BK = 1024


def _linear_kernel(x_ref, w_ref, b_ref, glh_ref, sq_ref, sqe_ref):
    gl = jnp.maximum(
        jnp.dot(x_ref[...], w_ref[...], preferred_element_type=jnp.float32)
        + b_ref[...],
        0.0,
    )
    glh_ref[...] = gl.astype(jnp.bfloat16)
    sq = jnp.sum(gl * gl, axis=1, keepdims=True)
    sq_ref[...] = sq
    sqe_ref[...] = sq + EPS


def _rowfold(p):
    acc = p[:, 0:128]
    for c in range(128, BN, 128):
        acc = acc + p[:, c:c + 128]
    return acc


def _prob_kernel(glh_ref, sqe_ref, sqt_ref, prob_ref, pb_ref, rs_ref):
    i = pl.program_id(0)
    j = pl.program_id(1)
    glr = glh_ref[pl.ds(i * BM, BM), :]
    glc = glh_ref[pl.ds(j * BN, BN), :]
    sq_r = sqe_ref[...]
    sq_c = sqt_ref[...]
    g2 = jax.lax.dot_general(
        glr * jnp.bfloat16(-2.0), glc, (((1,), (1,)), ((), ())),
        preferred_element_type=jnp.float32,
    )
    diff = jnp.maximum((g2 + sq_c) + sq_r, EPS)
    prob = jnp.exp(-jnp.sqrt(diff))

    hits_diag = jnp.logical_and(i * BM < (j + 1) * BN, j * BN < (i + 1) * BM)

    @pl.when(hits_diag)
    def _():
        rg = i * BM + jax.lax.broadcasted_iota(jnp.int32, (BM, BN), 0)
        cg = j * BN + jax.lax.broadcasted_iota(jnp.int32, (BM, BN), 1)
        pfix = jnp.where(rg == cg, 1.0, prob)
        prob_ref[...] = pfix
        pb_ref[...] = pfix.astype(jnp.bfloat16)
        ps = _rowfold(pfix)

        @pl.when(j == 0)
        def _():
            rs_ref[...] = ps

        @pl.when(j != 0)
        def _():
            rs_ref[...] += ps

    @pl.when(jnp.logical_not(hits_diag))
    def _():
        prob_ref[...] = prob
        pb_ref[...] = prob.astype(jnp.bfloat16)
        ps = _rowfold(prob)

        @pl.when(j == 0)
        def _():
            rs_ref[...] = ps

        @pl.when(j != 0)
        def _():
            rs_ref[...] += ps


def _dinv_kernel(rs_ref, x_ref, dinv_ref, uh_ref):
    deg = jnp.sum(rs_ref[...], axis=1, keepdims=True) - 1.0
    dinv = jnp.where(deg > 0.0, jax.lax.rsqrt(deg), 0.0)
    dinv_ref[...] = dinv
    uh_ref[...] = (dinv * x_ref[...]).astype(jnp.bfloat16)


def _tx1_kernel(pb_ref, uh_ref, dinv_ref, tx1_ref, u2h_ref, *, nj):
    j = pl.program_id(1)
    u_j = uh_ref[pl.ds(j * BK, BK), :]

    @pl.when(j == 0)
    def _():
        tx1_ref[...] = -uh_ref[pl.ds(pl.program_id(0) * BM, BM), :].astype(jnp.float32)

    tx1_ref[...] += jax.lax.dot_general(
        pb_ref[...], u_j, (((1,), (0,)), ((), ())),
        preferred_element_type=jnp.float32,
    )

    @pl.when(j == nj - 1)
    def _():
        dinv_i = dinv_ref[...]
        tx1 = -(dinv_i * tx1_ref[...])
        tx1_ref[...] = tx1
        u2h_ref[...] = (dinv_i * tx1).astype(jnp.bfloat16)


def _out_kernel(pb_ref, u2h_ref, tx1_ref, x_ref, dinv_ref, w_ref, b_ref,
                out_ref, acc_ref, *, nj):
    i = pl.program_id(0)
    j = pl.program_id(1)
    u2_j = u2h_ref[pl.ds(j * BK, BK), :]

    @pl.when(j == 0)
    def _():
        acc_ref[...] = -u2h_ref[pl.ds(i * BM, BM), :].astype(jnp.float32)

    acc_ref[...] += jax.lax.dot_general(
        pb_ref[...], u2_j, (((1,), (0,)), ((), ())),
        preferred_element_type=jnp.float32,
    )

    @pl.when(j == nj - 1)
    def _():
        lt = -(dinv_ref[...] * acc_ref[...])
        xi = x_ref[pl.ds(i * BM, BM), :]
        tx1i = tx1_ref[pl.ds(i * BM, BM), :]
        tx2 = 2.0 * lt - xi
        out = jnp.dot(xi, w_ref[0], preferred_element_type=jnp.float32)
        out += jnp.dot(tx1i, w_ref[1], preferred_element_type=jnp.float32)
        out += jnp.dot(tx2, w_ref[2], preferred_element_type=jnp.float32)
        out_ref[...] = jnp.maximum(out + b_ref[...], 0.0)


def kernel(input, adj, lin_w, lin_b, cheb_w, cheb_b):
    x = input
    n, d = x.shape
    ni, nj = n // BM, n // BN
    njk = n // BK

    glh, sq, sqe = pl.pallas_call(
        _linear_kernel,
        grid=(n // BL,),
        in_specs=[
            pl.BlockSpec((BL, d), lambda i: (i, 0)),
            pl.BlockSpec((d, d), lambda i: (0, 0)),
            pl.BlockSpec((1, d), lambda i: (0, 0)),
        ],
        out_specs=[
            pl.BlockSpec((BL, d), lambda i: (i, 0)),
            pl.BlockSpec((BL, 1), lambda i: (i, 0)),
            pl.BlockSpec((BL, 1), lambda i: (i, 0)),
        ],
        out_shape=[
            jax.ShapeDtypeStruct((n, d), jnp.bfloat16),
            jax.ShapeDtypeStruct((n, 1), jnp.float32),
            jax.ShapeDtypeStruct((n, 1), jnp.float32),
        ],
        compiler_params=pltpu.CompilerParams(
            dimension_semantics=("parallel",)),
    )(x, lin_w, lin_b.reshape(1, d))

    sqt = sq.reshape(1, n)

    prob, pb, rs = pl.pallas_call(
        _prob_kernel,
        grid=(ni, nj),
        in_specs=[
            pl.BlockSpec((n, d), lambda i, j: (0, 0)),
            pl.BlockSpec((BM, 1), lambda i, j: (i, 0)),
            pl.BlockSpec((1, BN), lambda i, j: (0, j)),
        ],
        out_specs=[
            pl.BlockSpec((BM, BN), lambda i, j: (i, j)),
            pl.BlockSpec((BM, BN), lambda i, j: (i, j)),
            pl.BlockSpec((BM, 128), lambda i, j: (i, 0)),
        ],
        out_shape=[
            jax.ShapeDtypeStruct((n, n), jnp.float32),
            jax.ShapeDtypeStruct((n, n), jnp.bfloat16),
            jax.ShapeDtypeStruct((n, 128), jnp.float32),
        ],
        compiler_params=pltpu.CompilerParams(
            dimension_semantics=("parallel", "arbitrary")),
    )(glh, sqe, sqt)

    dinv, uh = pl.pallas_call(
        _dinv_kernel,
        grid=(ni,),
        in_specs=[
            pl.BlockSpec((BM, 128), lambda i: (i, 0)),
            pl.BlockSpec((BM, d), lambda i: (i, 0)),
        ],
        out_specs=[
            pl.BlockSpec((BM, 1), lambda i: (i, 0)),
            pl.BlockSpec((BM, d), lambda i: (i, 0)),
        ],
        out_shape=[
            jax.ShapeDtypeStruct((n, 1), jnp.float32),
            jax.ShapeDtypeStruct((n, d), jnp.bfloat16),
        ],
        compiler_params=pltpu.CompilerParams(
            dimension_semantics=("parallel",)),
    )(rs, x)

    tx1, u2h = pl.pallas_call(
        functools.partial(_tx1_kernel, nj=njk),
        grid=(ni, njk),
        in_specs=[
            pl.BlockSpec((BM, BK), lambda i, j: (i, j)),
            pl.BlockSpec((n, d), lambda i, j: (0, 0)),
            pl.BlockSpec((BM, 1), lambda i, j: (i, 0)),
        ],
        out_specs=[
            pl.BlockSpec((BM, d), lambda i, j: (i, 0)),
            pl.BlockSpec((BM, d), lambda i, j: (i, 0)),
        ],
        out_shape=[
            jax.ShapeDtypeStruct((n, d), jnp.float32),
            jax.ShapeDtypeStruct((n, d), jnp.bfloat16),
        ],
        compiler_params=pltpu.CompilerParams(
            dimension_semantics=("parallel", "arbitrary")),
    )(pb, uh, dinv)

    out = pl.pallas_call(
        functools.partial(_out_kernel, nj=njk),
        grid=(ni, njk),
        in_specs=[
            pl.BlockSpec((BM, BK), lambda i, j: (i, j)),
            pl.BlockSpec((n, d), lambda i, j: (0, 0)),
            pl.BlockSpec((n, d), lambda i, j: (0, 0)),
            pl.BlockSpec((n, d), lambda i, j: (0, 0)),
            pl.BlockSpec((BM, 1), lambda i, j: (i, 0)),
            pl.BlockSpec((3, d, d), lambda i, j: (0, 0, 0)),
            pl.BlockSpec((1, d), lambda i, j: (0, 0)),
        ],
        out_specs=pl.BlockSpec((BM, d), lambda i, j: (i, 0)),
        out_shape=jax.ShapeDtypeStruct((n, d), jnp.float32),
        scratch_shapes=[pltpu.VMEM((BM, d), jnp.float32)],
        compiler_params=pltpu.CompilerParams(
            dimension_semantics=("parallel", "arbitrary")),
    )(pb, u2h, tx1, x, dinv, cheb_w, cheb_b.reshape(1, d))

    return out, prob


# speedup vs baseline: 2.0965x; 1.3122x over previous
import functools

import jax
import jax.numpy as jnp
from jax.experimental import pallas as pl
from jax.experimental.pallas import tpu as pltpu

EPS = 1.1920929e-07

BL = 512
BM = 512
BN = 1024
BK = 1024


def _linear_kernel(x_ref, w_ref, b_ref, glh_ref, sq_ref, sqe_ref):
    gl = jnp.maximum(
        jnp.dot(x_ref[...], w_ref[...], preferred_element_type=jnp.float32)
        + b_ref[...],
        0.0,
    )
    glh_ref[...] = gl.astype(jnp.bfloat16)
    sq = jnp.sum(gl * gl, axis=1, keepdims=True)
    sq_ref[...] = sq
    sqe_ref[...] = sq + EPS


def _rowfold(p):
    acc = p[:, 0:128]
    for c in range(128, BN, 128):
        acc = acc + p[:, c:c + 128]
    return acc


def _prob_kernel(glh_ref, sqe_ref, sqt_ref, prob_ref, pb_ref, rs_ref):
    i = pl.program_id(0)
    j = pl.program_id(1)
    glr = glh_ref[pl.ds(i * BM, BM), :]
    glc = glh_ref[pl.ds(j * BN, BN), :]
    sq_r = sqe_ref[...]
    sq_c = sqt_ref[...]
    g2 = jax.lax.dot_general(
        glr * jnp.bfloat16(-2.0), glc, (((1,), (1,)), ((), ())),
        preferred_element_type=jnp.float32,
    )
    diff = jnp.maximum((g2 + sq_c) + sq_r, EPS)
    prob = jnp.exp2((diff * jnp.float32(-1.4426950408889634))
                    * jax.lax.rsqrt(diff))

    hits_diag = jnp.logical_and(i * BM < (j + 1) * BN, j * BN < (i + 1) * BM)

    @pl.when(hits_diag)
    def _():
        rg = i * BM + jax.lax.broadcasted_iota(jnp.int32, (BM, BN), 0)
        cg = j * BN + jax.lax.broadcasted_iota(jnp.int32, (BM, BN), 1)
        pfix = jnp.where(rg == cg, 1.0, prob)
        prob_ref[...] = pfix
        pb_ref[...] = pfix.astype(jnp.bfloat16)
        ps = _rowfold(pfix)

        @pl.when(j == 0)
        def _():
            rs_ref[...] = ps

        @pl.when(j != 0)
        def _():
            rs_ref[...] += ps

    @pl.when(jnp.logical_not(hits_diag))
    def _():
        prob_ref[...] = prob
        pb_ref[...] = prob.astype(jnp.bfloat16)
        ps = _rowfold(prob)

        @pl.when(j == 0)
        def _():
            rs_ref[...] = ps

        @pl.when(j != 0)
        def _():
            rs_ref[...] += ps


def _dinv_kernel(rs_ref, x_ref, dinv_ref, uh_ref):
    deg = jnp.sum(rs_ref[...], axis=1, keepdims=True) - 1.0
    dinv = jnp.where(deg > 0.0, jax.lax.rsqrt(deg), 0.0)
    dinv_ref[...] = dinv
    uh_ref[...] = (dinv * x_ref[...]).astype(jnp.bfloat16)


def _tx1_kernel(pb_ref, uh_ref, dinv_ref, tx1_ref, u2h_ref):
    i = pl.program_id(0)
    z = jax.lax.dot_general(
        pb_ref[...], uh_ref[...], (((1,), (0,)), ((), ())),
        preferred_element_type=jnp.float32,
    )
    u_i = uh_ref[pl.ds(i * BM, BM), :].astype(jnp.float32)
    dinv_i = dinv_ref[...]
    tx1 = -(dinv_i * (z - u_i))
    tx1_ref[...] = tx1
    u2h_ref[...] = (dinv_i * tx1).astype(jnp.bfloat16)


def _out_kernel(pb_ref, u2h_ref, tx1_ref, x_ref, dinv_ref, w_ref, b_ref,
                out_ref):
    i = pl.program_id(0)
    z = jax.lax.dot_general(
        pb_ref[...], u2h_ref[...], (((1,), (0,)), ((), ())),
        preferred_element_type=jnp.float32,
    )
    u2_i = u2h_ref[pl.ds(i * BM, BM), :].astype(jnp.float32)
    lt = -(dinv_ref[...] * (z - u2_i))
    xi = x_ref[pl.ds(i * BM, BM), :]
    tx1i = tx1_ref[pl.ds(i * BM, BM), :]
    tx2 = 2.0 * lt - xi
    out = jnp.dot(xi, w_ref[0], preferred_element_type=jnp.float32)
    out += jnp.dot(tx1i, w_ref[1], preferred_element_type=jnp.float32)
    out += jnp.dot(tx2, w_ref[2], preferred_element_type=jnp.float32)
    out_ref[...] = jnp.maximum(out + b_ref[...], 0.0)


def kernel(input, adj, lin_w, lin_b, cheb_w, cheb_b):
    x = input
    n, d = x.shape
    ni, nj = n // BM, n // BN
    njk = n // BK

    glh, sq, sqe = pl.pallas_call(
        _linear_kernel,
        grid=(n // BL,),
        in_specs=[
            pl.BlockSpec((BL, d), lambda i: (i, 0)),
            pl.BlockSpec((d, d), lambda i: (0, 0)),
            pl.BlockSpec((1, d), lambda i: (0, 0)),
        ],
        out_specs=[
            pl.BlockSpec((BL, d), lambda i: (i, 0)),
            pl.BlockSpec((BL, 1), lambda i: (i, 0)),
            pl.BlockSpec((BL, 1), lambda i: (i, 0)),
        ],
        out_shape=[
            jax.ShapeDtypeStruct((n, d), jnp.bfloat16),
            jax.ShapeDtypeStruct((n, 1), jnp.float32),
            jax.ShapeDtypeStruct((n, 1), jnp.float32),
        ],
        compiler_params=pltpu.CompilerParams(
            dimension_semantics=("parallel",)),
    )(x, lin_w, lin_b.reshape(1, d))

    sqt = sq.reshape(1, n)

    prob, pb, rs = pl.pallas_call(
        _prob_kernel,
        grid=(ni, nj),
        in_specs=[
            pl.BlockSpec((n, d), lambda i, j: (0, 0)),
            pl.BlockSpec((BM, 1), lambda i, j: (i, 0)),
            pl.BlockSpec((1, BN), lambda i, j: (0, j)),
        ],
        out_specs=[
            pl.BlockSpec((BM, BN), lambda i, j: (i, j)),
            pl.BlockSpec((BM, BN), lambda i, j: (i, j)),
            pl.BlockSpec((BM, 128), lambda i, j: (i, 0)),
        ],
        out_shape=[
            jax.ShapeDtypeStruct((n, n), jnp.float32),
            jax.ShapeDtypeStruct((n, n), jnp.bfloat16),
            jax.ShapeDtypeStruct((n, 128), jnp.float32),
        ],
        compiler_params=pltpu.CompilerParams(
            dimension_semantics=("parallel", "arbitrary")),
    )(glh, sqe, sqt)

    dinv, uh = pl.pallas_call(
        _dinv_kernel,
        grid=(ni,),
        in_specs=[
            pl.BlockSpec((BM, 128), lambda i: (i, 0)),
            pl.BlockSpec((BM, d), lambda i: (i, 0)),
        ],
        out_specs=[
            pl.BlockSpec((BM, 1), lambda i: (i, 0)),
            pl.BlockSpec((BM, d), lambda i: (i, 0)),
        ],
        out_shape=[
            jax.ShapeDtypeStruct((n, 1), jnp.float32),
            jax.ShapeDtypeStruct((n, d), jnp.bfloat16),
        ],
        compiler_params=pltpu.CompilerParams(
            dimension_semantics=("parallel",)),
    )(rs, x)

    tx1, u2h = pl.pallas_call(
        _tx1_kernel,
        grid=(ni,),
        in_specs=[
            pl.BlockSpec((BM, n), lambda i: (i, 0)),
            pl.BlockSpec((n, d), lambda i: (0, 0)),
            pl.BlockSpec((BM, 1), lambda i: (i, 0)),
        ],
        out_specs=[
            pl.BlockSpec((BM, d), lambda i: (i, 0)),
            pl.BlockSpec((BM, d), lambda i: (i, 0)),
        ],
        out_shape=[
            jax.ShapeDtypeStruct((n, d), jnp.float32),
            jax.ShapeDtypeStruct((n, d), jnp.bfloat16),
        ],
        compiler_params=pltpu.CompilerParams(
            dimension_semantics=("arbitrary",)),
    )(pb, uh, dinv)

    out = pl.pallas_call(
        _out_kernel,
        grid=(ni,),
        in_specs=[
            pl.BlockSpec((BM, n), lambda i: (i, 0)),
            pl.BlockSpec((n, d), lambda i: (0, 0)),
            pl.BlockSpec((n, d), lambda i: (0, 0)),
            pl.BlockSpec((n, d), lambda i: (0, 0)),
            pl.BlockSpec((BM, 1), lambda i: (i, 0)),
            pl.BlockSpec((3, d, d), lambda i: (0, 0, 0)),
            pl.BlockSpec((1, d), lambda i: (0, 0)),
        ],
        out_specs=pl.BlockSpec((BM, d), lambda i: (i, 0)),
        out_shape=jax.ShapeDtypeStruct((n, d), jnp.float32),
        compiler_params=pltpu.CompilerParams(
            dimension_semantics=("arbitrary",)),
    )(pb, u2h, tx1, x, dinv, cheb_w, cheb_b.reshape(1, d))

    return out, prob
